# Initial kernel scaffold; baseline (speedup 1.0000x reference)
import jax
import jax.numpy as jnp
from jax.experimental import pallas as pl


def kernel(self_loop, x_queue, a_queue, obs_queue, obs_a_queue, u_gamma_queue, sage_lin_l_w, sage_lin_l_b, sage_lin_r_w, lstm1_w_ih, lstm1_w_hh, lstm1_b_ih, lstm1_b_hh, lstm2_w_ih, lstm2_w_hh, lstm2_b_ih, lstm2_b_hh, lin_w, lin_b, lin1_w, lin1_b):
    raise NotImplementedError("write your pallas kernel here")


# speedup vs baseline: 1.3274x; 1.3274x over previous
import jax
import jax.numpy as jnp
from jax.experimental import pallas as pl
from jax.experimental.pallas import tpu as pltpu

_K = 3
_L = 2
_H = 64
_H2 = 72
_N = 2048
_T = _L + _K - 1
_F32 = jnp.float32


def _dot(a, b):
    return jax.lax.dot_general(a, b, (((1,), (0,)), ((), ())),
                               preferred_element_type=_F32)


def _body(xT_ref, a_ref, ou_ref,
          wl_ref, bl_ref, wr_ref,
          w1x_ref, w1h_ref, b1_ref,
          w2a_ref, w2o_ref, w2h_ref, b2_ref,
          linw_ref, linb_ref, lin1w_ref, lin1b_ref,
          out_ref,
          d0, d1, d2, h2s, c2s):
    i = pl.program_id(0)
    a = a_ref[0]
    xT = xT_ref[0]

    @pl.when(i > 0)
    def _merged():
        s = jnp.sum(a, axis=1, keepdims=True)
        inv_s = 1.0 / s
        m0 = _dot(a, d1[...] * inv_s)
        m1 = _dot(a, d2[...] * inv_s)
        d0[...] = m0
        d1[...] = m1

    @pl.when(i == 0)
    def _init():
        d0[...] = jnp.zeros((_N, _H), _F32)
        d1[...] = jnp.zeros((_N, _H), _F32)

    deg = jnp.sum(a, axis=0, keepdims=True)
    inv_deg = 1.0 / jnp.maximum(deg, 1.0)
    aggT = _dot(xT, a) * inv_deg
    outT = _dot(wl_ref[...], aggT) + bl_ref[...] + _dot(wr_ref[...], xT)
    nsq = jnp.sum(outT * outT, axis=0, keepdims=True)
    inv_n = 1.0 / jnp.maximum(jnp.sqrt(nsq), 1e-12)
    xnT = jnp.maximum(outT * inv_n, 0.0)
    d2[...] = xnT.T

    @pl.when(i >= _K - 1)
    def _tail():
        h = jnp.zeros((_N, _H), _F32)
        c = jnp.zeros((_N, _H), _F32)
        for xt in (d0[...], d1[...], d2[...]):
            ig = jax.nn.sigmoid(_dot(xt, w1x_ref[0]) + _dot(h, w1h_ref[0]) + b1_ref[0])
            fg = jax.nn.sigmoid(_dot(xt, w1x_ref[1]) + _dot(h, w1h_ref[1]) + b1_ref[1])
            gg = jnp.tanh(_dot(xt, w1x_ref[2]) + _dot(h, w1h_ref[2]) + b1_ref[2])
            og = jax.nn.sigmoid(_dot(xt, w1x_ref[3]) + _dot(h, w1h_ref[3]) + b1_ref[3])
            c = fg * c + ig * gg
            h = og * jnp.tanh(c)

        ou = ou_ref[0]
        first = i == _K - 1
        h2 = jnp.where(first, 0.0, h2s[...])
        c2 = jnp.where(first, 0.0, c2s[...])
        gs = []
        for g in range(4):
            gs.append(_dot(h, w2a_ref[g])
                      + jax.lax.dot_general(ou, w2o_ref[g], (((0,), (0,)), ((), ())),
                                            preferred_element_type=_F32)
                      + _dot(h2, w2h_ref[g]) + b2_ref[g])
        ig = jax.nn.sigmoid(gs[0])
        fg = jax.nn.sigmoid(gs[1])
        gg = jnp.tanh(gs[2])
        og = jax.nn.sigmoid(gs[3])
        c2 = fg * c2 + ig * gg
        h2 = og * jnp.tanh(c2)
        h2s[...] = h2
        c2s[...] = c2

        @pl.when(i == _T - 1)
        def _final():
            xl = jnp.maximum(_dot(h2, linw_ref[...]) + linb_ref[...], 0.0)
            out_ref[...] = _dot(xl, lin1w_ref[...]) + lin1b_ref[...]


def kernel(self_loop, x_queue, a_queue, obs_queue, obs_a_queue, u_gamma_queue,
           sage_lin_l_w, sage_lin_l_b, sage_lin_r_w,
           lstm1_w_ih, lstm1_w_hh, lstm1_b_ih, lstm1_b_hh,
           lstm2_w_ih, lstm2_w_hh, lstm2_b_ih, lstm2_b_hh,
           lin_w, lin_b, lin1_w, lin1_b):
    del self_loop, obs_a_queue

    xT_q = x_queue.transpose(0, 2, 1)
    ou_q = jnp.concatenate([obs_queue.transpose(0, 2, 1),
                            u_gamma_queue.transpose(0, 2, 1)], axis=1)
    bl = sage_lin_l_b.reshape(_H, 1)

    w1x = lstm1_w_ih.reshape(4, _H, _H).transpose(0, 2, 1)
    w1h = lstm1_w_hh.reshape(4, _H, _H).transpose(0, 2, 1)
    b1 = (lstm1_b_ih + lstm1_b_hh).reshape(4, 1, _H)

    w2i = lstm2_w_ih.reshape(4, _H2, _H2)
    w2a = w2i[:, :, :_H].transpose(0, 2, 1)
    w2o = w2i[:, :, _H:].transpose(0, 2, 1)
    w2h = lstm2_w_hh.reshape(4, _H2, _H2).transpose(0, 2, 1)
    b2 = (lstm2_b_ih + lstm2_b_hh).reshape(4, 1, _H2)

    linwT = lin_w.T
    linb = lin_b.reshape(1, _H2)
    lin1wT = lin1_w.T
    lin1b = lin1_b.reshape(1, 2)

    full = lambda shape: pl.BlockSpec(shape, lambda i: (0,) * len(shape))
    grid_spec = pltpu.PrefetchScalarGridSpec(
        num_scalar_prefetch=0,
        grid=(_T,),
        in_specs=[
            pl.BlockSpec((1, 6, _N), lambda i: (i, 0, 0)),
            pl.BlockSpec((1, _N, _N), lambda i: (i, 0, 0)),
            pl.BlockSpec((1, 8, _N), lambda i: (i, 0, 0)),
            full((_H, 6)), full((_H, 1)), full((_H, 6)),
            full((4, _H, _H)), full((4, _H, _H)), full((4, 1, _H)),
            full((4, _H, _H2)), full((4, 8, _H2)),
            full((4, _H2, _H2)), full((4, 1, _H2)),
            full((_H2, _H2)), full((1, _H2)), full((_H2, 2)), full((1, 2)),
        ],
        out_specs=pl.BlockSpec((_N, 2), lambda i: (0, 0)),
        scratch_shapes=[
            pltpu.VMEM((_N, _H), _F32),
            pltpu.VMEM((_N, _H), _F32),
            pltpu.VMEM((_N, _H), _F32),
            pltpu.VMEM((_N, _H2), _F32),
            pltpu.VMEM((_N, _H2), _F32),
        ],
    )

    return pl.pallas_call(
        _body,
        grid_spec=grid_spec,
        out_shape=jax.ShapeDtypeStruct((_N, 2), _F32),
        compiler_params=pltpu.CompilerParams(
            dimension_semantics=("arbitrary",),
        ),
    )(xT_q, a_queue, ou_q,
      sage_lin_l_w, bl, sage_lin_r_w,
      w1x, w1h, b1,
      w2a, w2o, w2h, b2,
      linwT, linb, lin1wT, lin1b)
